# baseline (speedup 1.0000x reference)
import functools

import jax
import jax.numpy as jnp
from jax import lax
from jax.experimental import pallas as pl
from jax.experimental.pallas import tpu as pltpu
from jax.experimental.pallas import tpu_sc as plsc

N0, N1, N2, D = 50000, 10000, 2048, 128
E1, E2 = 160000, 32768
TEMPER, WEIGHT = 0.2, 1.0

NC, NS = 2, 16
NW = NC * NS
B = 128
R1P = 1280
R2 = E2 // B
PAD1 = R1P * B - E1

H0 = 51200
H1 = 10240
H2 = 2048
N1P = 10240

_MESH = plsc.VectorSubcoreMesh(core_axis_name="c", subcore_axis_name="s")


def _fill(ref, rows, val):
    v = jnp.full((16,), val, jnp.float32)
    if len(ref.shape) == 1:
        for i in range(rows // 16):
            ref[pl.ds(i * 16, 16)] = v
    else:
        for r in range(rows):
            for j in range(ref.shape[1] // 16):
                ref[r, pl.ds(j * 16, 16)] = v


@functools.partial(
    pl.kernel,
    out_type=(
        jax.ShapeDtypeStruct((NC * H0,), jnp.float32),
        jax.ShapeDtypeStruct((NC * H1,), jnp.float32),
        jax.ShapeDtypeStruct((NC * H1,), jnp.float32),
        jax.ShapeDtypeStruct((NC * H2,), jnp.float32),
    ),
    mesh=_MESH,
    scratch_types=[
        pltpu.VMEM((3200,), jnp.float32),
        pltpu.VMEM((B,), jnp.float32),
        pltpu.VMEM((R1P // NW, B), jnp.int32),
        pltpu.VMEM((R1P // NW, B), jnp.int32),
        pltpu.VMEM((R2 // NW, B), jnp.int32),
        pltpu.VMEM((R2 // NW, B), jnp.int32),
        pltpu.SemaphoreType.DMA,
        pltpu.VMEM_SHARED((H0,), jnp.float32),
        pltpu.VMEM_SHARED((H1,), jnp.float32),
        pltpu.VMEM_SHARED((H1,), jnp.float32),
        pltpu.VMEM_SHARED((H2,), jnp.float32),
    ],
)
def _hists(s1, d1, s2, d2, o0, o1, o2, o3, zb, ones, i1, i2, i3, i4,
           sem, h0, h1a, h1b, h3):
    c = lax.axis_index("c")
    s = lax.axis_index("s")
    wid = s * NC + c
    nj1 = R1P // NW
    nj2 = R2 // NW

    _fill(zb, 3200, 0.0)
    _fill(ones, B, 1.0)
    pltpu.sync_copy(zb.at[pl.ds(0, 3200)], h0.at[pl.ds(s * 3200, 3200)])
    pltpu.sync_copy(zb.at[pl.ds(0, 640)], h1a.at[pl.ds(s * 640, 640)])
    pltpu.sync_copy(zb.at[pl.ds(0, 640)], h1b.at[pl.ds(s * 640, 640)])
    pltpu.sync_copy(zb.at[pl.ds(0, 128)], h3.at[pl.ds(s * 128, 128)])
    pltpu.sync_copy(s1.at[pl.ds(pl.multiple_of(wid * nj1, 8), nj1)], i1)
    pltpu.sync_copy(d1.at[pl.ds(pl.multiple_of(wid * nj1, 8), nj1)], i2)
    pltpu.sync_copy(s2.at[pl.ds(pl.multiple_of(wid * nj2, 8), nj2)], i3)
    pltpu.sync_copy(d2.at[pl.ds(pl.multiple_of(wid * nj2, 8), nj2)], i4)
    plsc.subcore_barrier()

    work = ([(i1, h0, j) for j in range(nj1)]
            + [(i2, h1a, j) for j in range(nj1)]
            + [(i3, h1b, j) for j in range(nj2)]
            + [(i4, h3, j) for j in range(nj2)])
    K = 8
    for g in range(0, len(work), K):
        descs = [
            pltpu.async_copy(ones, hist.at[idx.at[j]], sem, add=True)
            for idx, hist, j in work[g:g + K]
        ]
        for dsc in descs:
            dsc.wait()
    plsc.subcore_barrier()

    pltpu.sync_copy(h0.at[pl.ds(pl.multiple_of(s * 3200, 128), 3200)],
                    o0.at[pl.ds(pl.multiple_of(c * H0 + s * 3200, 128), 3200)])
    pltpu.sync_copy(h1a.at[pl.ds(pl.multiple_of(s * 640, 128), 640)],
                    o1.at[pl.ds(pl.multiple_of(c * H1 + s * 640, 128), 640)])
    pltpu.sync_copy(h1b.at[pl.ds(pl.multiple_of(s * 640, 128), 640)],
                    o2.at[pl.ds(pl.multiple_of(c * H1 + s * 640, 128), 640)])
    pltpu.sync_copy(h3.at[pl.ds(pl.multiple_of(s * 128, 128), 128)],
                    o3.at[pl.ds(pl.multiple_of(c * H2 + s * 128, 128), 128)])


def _make_agg(n_dst, n_chunks, two_tables):
    rows_per_tile = n_dst // NS
    nj = n_chunks // NW

    n_out = 2 if two_tables else 1
    out_type = tuple(
        jax.ShapeDtypeStruct((NC * n_dst, D), jnp.float32)
        for _ in range(n_out)
    )
    if not two_tables:
        out_type = out_type[0]
    nbuf = 2
    scratch = (
        [pltpu.VMEM((nj, B), jnp.int32),
         pltpu.VMEM((nj, B), jnp.int32)]
        + [pltpu.VMEM((B, D), jnp.float32)] * (nbuf * (2 if two_tables else 1))
        + [pltpu.SemaphoreType.DMA] * (2 * nbuf)
        + [pltpu.VMEM_SHARED((n_dst, D), jnp.float32) for _ in range(n_out)]
    )

    @functools.partial(
        pl.kernel, out_type=out_type, mesh=_MESH, scratch_types=scratch
    )
    def agg(*refs):
        if two_tables:
            (ta, tb, se, de, oa, ob, idxs, idxd, ra0, ra1, rb0, rb1,
             sem0, sem1, semx0, semx1, acca, accb) = refs
            rb = (rb0, rb1)
            ra = (ra0, ra1)
            sems = (sem0, sem1)
        else:
            (ta, se, de, oa, idxs, idxd, ra0, ra1,
             sg0, sg1, ss0, ss1, acca) = refs
            ra = (ra0, ra1)
            sg = (sg0, sg1)
            ss = (ss0, ss1)
        c = lax.axis_index("c")
        s = lax.axis_index("s")
        wid = s * NC + c

        pltpu.sync_copy(se.at[pl.ds(pl.multiple_of(wid * nj, 8), nj)], idxs)
        pltpu.sync_copy(de.at[pl.ds(pl.multiple_of(wid * nj, 8), nj)], idxd)
        _fill(ra0, B, 0.0)
        for k in range(rows_per_tile // 128):
            base = pl.multiple_of(s * rows_per_tile + k * 128, 8)
            pltpu.sync_copy(ra0, acca.at[pl.ds(base, 128)])
            if two_tables:
                pltpu.sync_copy(ra0, accb.at[pl.ds(base, 128)])
        plsc.subcore_barrier()

        if two_tables:
            ga = {0: pltpu.async_copy(ta.at[idxs.at[0]], ra[0], sems[0])}
            for j in range(nj):
                ga[j].wait()
                gb = pltpu.async_copy(tb.at[idxs.at[j]], rb[j % 2], sems[j % 2])
                if j + 1 < nj:
                    ga[j + 1] = pltpu.async_copy(
                        ta.at[idxs.at[j + 1]], ra[(j + 1) % 2],
                        sems[(j + 1) % 2])
                pltpu.sync_copy(ra[j % 2], acca.at[idxd.at[j]], add=True)
                gb.wait()
                pltpu.sync_copy(rb[j % 2], accb.at[idxd.at[j]], add=True)
        else:
            ga, sc = {}, {}
            ga[0] = pltpu.async_copy(ta.at[idxs.at[0]], ra[0], sg[0])
            for j in range(nj):
                ga[j].wait()
                sc[j] = pltpu.async_copy(ra[j % 2], acca.at[idxd.at[j]],
                                         ss[j % 2], add=True)
                if j + 1 < nj:
                    if j >= 1:
                        sc[j - 1].wait()
                    ga[j + 1] = pltpu.async_copy(
                        ta.at[idxs.at[j + 1]], ra[(j + 1) % 2],
                        sg[(j + 1) % 2])
            sc[nj - 2].wait()
            sc[nj - 1].wait()
        plsc.subcore_barrier()

        src_base = pl.multiple_of(s * rows_per_tile, 8)
        dst_base = pl.multiple_of(c * n_dst + s * rows_per_tile, 8)
        pltpu.sync_copy(
            acca.at[pl.ds(src_base, rows_per_tile)],
            oa.at[pl.ds(dst_base, rows_per_tile)],
        )
        if two_tables:
            pltpu.sync_copy(
                accb.at[pl.ds(src_base, rows_per_tile)],
                ob.at[pl.ds(dst_base, rows_per_tile)],
            )

    return agg


_agg1 = _make_agg(N1P, R1P, two_tables=False)
_agg2 = _make_agg(N2, R2, two_tables=True)


def _prescale_body(x_ref, dp_ref, o_ref):
    blk = x_ref.shape[0]
    row = (pl.program_id(0) * blk
           + lax.broadcasted_iota(jnp.int32, (blk, 1), 0))
    deg = dp_ref[0] + dp_ref[1] - jnp.where(row < PAD1, 1.0, 0.0)
    rs = lax.rsqrt(jnp.maximum(deg, 1.0))
    o_ref[...] = x_ref[...] * rs


def _prescale(x0, deg_parts):
    blk = 2000
    return pl.pallas_call(
        _prescale_body,
        grid=(N0 // blk,),
        in_specs=[
            pl.BlockSpec((blk, D), lambda i: (i, 0)),
            pl.BlockSpec((NC, blk, 1), lambda i: (0, i, 0)),
        ],
        out_specs=pl.BlockSpec((blk, D), lambda i: (i, 0)),
        out_shape=jax.ShapeDtypeStruct((N0, D), jnp.float32),
    )(x0, deg_parts)


def _layer1_body(agg_ref, din_ref, dout_ref, w_ref, b_ref, h_ref, hs_ref):
    agg = agg_ref[0] + agg_ref[1]
    rs_in = lax.rsqrt(jnp.maximum(din_ref[0] + din_ref[1], 1.0))
    h = jnp.dot(agg * rs_in, w_ref[...], preferred_element_type=jnp.float32)
    h = h + b_ref[...]
    h_ref[...] = h
    rs_out = lax.rsqrt(jnp.maximum(dout_ref[0] + dout_ref[1], 1.0))
    hs_ref[...] = h * rs_out


def _layer1(agg_parts, din_parts, dout_parts, W1, b1):
    blk = 1280
    return pl.pallas_call(
        _layer1_body,
        grid=(N1P // blk,),
        in_specs=[
            pl.BlockSpec((NC, blk, D), lambda i: (0, i, 0)),
            pl.BlockSpec((NC, blk, 1), lambda i: (0, i, 0)),
            pl.BlockSpec((NC, blk, 1), lambda i: (0, i, 0)),
            pl.BlockSpec((D, D), lambda i: (0, 0)),
            pl.BlockSpec((1, D), lambda i: (0, 0)),
        ],
        out_specs=[
            pl.BlockSpec((blk, D), lambda i: (i, 0)),
            pl.BlockSpec((blk, D), lambda i: (i, 0)),
        ],
        out_shape=[
            jax.ShapeDtypeStruct((N1P, D), jnp.float32),
            jax.ShapeDtypeStruct((N1P, D), jnp.float32),
        ],
    )(agg_parts, din_parts, dout_parts, W1, b1)


def _norm_rows(x):
    n = jnp.sqrt(jnp.sum(x * x, axis=1, keepdims=True))
    return x / jnp.maximum(n, 1e-12)


def _feats_body(agg_ref, nbs_ref, cnt_ref, w2_ref, b2_ref, p1w_ref, p1b_ref,
                p2w_ref, p2b_ref, proj_ref, moco_ref, nb_ref):
    cnt = jnp.maximum(cnt_ref[0] + cnt_ref[1], 1.0)
    agg = (agg_ref[0] + agg_ref[1]) * lax.rsqrt(cnt)
    conv = jnp.dot(agg, w2_ref[...], preferred_element_type=jnp.float32)
    conv = conv + b2_ref[...]
    moco = _norm_rows(conv)
    moco_ref[...] = moco
    nb_ref[...] = _norm_rows((nbs_ref[0] + nbs_ref[1]) / cnt)
    h = jnp.maximum(
        jnp.dot(moco, p1w_ref[...], preferred_element_type=jnp.float32)
        + p1b_ref[...], 0.0)
    h = jnp.maximum(
        jnp.dot(h, p2w_ref[...], preferred_element_type=jnp.float32)
        + p2b_ref[...], 0.0)
    p = jnp.dot(h, p2w_ref[...], preferred_element_type=jnp.float32)
    p = p + p2b_ref[...]
    proj_ref[...] = _norm_rows(p)


def _feats(agg_parts, nb_parts, cnt_parts, W2, b2, P1W, P1b, P2W, P2b):
    return pl.pallas_call(
        _feats_body,
        out_shape=[
            jax.ShapeDtypeStruct((N2, D), jnp.float32),
            jax.ShapeDtypeStruct((N2, D), jnp.float32),
            jax.ShapeDtypeStruct((N2, D), jnp.float32),
        ],
    )(agg_parts, nb_parts, cnt_parts, W2, b2, P1W, P1b, P2W, P2b)


def _loss_body(p_ref, m_ref, nb_ref, o_ref):
    i = pl.program_id(0)
    blk = p_ref.shape[0]
    p = p_ref[...]

    def nce(bmat):
        sim = lax.dot_general(
            p, bmat, (((1,), (1,)), ((), ())),
            preferred_element_type=jnp.float32) / TEMPER
        e = jnp.exp(sim)
        rsum = jnp.sum(e, axis=1)
        col = lax.broadcasted_iota(jnp.int32, sim.shape, 1)
        row = lax.broadcasted_iota(jnp.int32, sim.shape, 0)
        diag = jnp.sum(jnp.where(col == row + i * blk, e, 0.0), axis=1)
        return jnp.sum(-jnp.log(diag / rsum))

    part = (nce(m_ref[...]) + WEIGHT * nce(nb_ref[...])) / N2

    @pl.when(i == 0)
    def _():
        o_ref[...] = jnp.zeros_like(o_ref)

    o_ref[...] = o_ref[...] + part


def _loss(proj, moco, nbn):
    blk = 256
    return pl.pallas_call(
        _loss_body,
        grid=(N2 // blk,),
        in_specs=[
            pl.BlockSpec((blk, D), lambda i: (i, 0)),
            pl.BlockSpec((N2, D), lambda i: (0, 0)),
            pl.BlockSpec((N2, D), lambda i: (0, 0)),
        ],
        out_specs=pl.BlockSpec((1, 1), lambda i: (0, 0)),
        out_shape=jax.ShapeDtypeStruct((1, 1), jnp.float32),
    )(proj, moco, nbn)


def kernel(x0, src1, dst1, src2, dst2, W1, b1, W2, b2, P1W, P1b, P2W, P2b):
    it = jnp.arange(PAD1, dtype=jnp.int32)
    src1i = src1.astype(jnp.int32)
    dst1i = dst1.astype(jnp.int32)
    s1a = jnp.concatenate([src1i, it]).reshape(R1P, B)
    d1p = jnp.concatenate(
        [dst1i, N1 + it % (N1P - N1)]).reshape(R1P, B)
    s2 = src2.astype(jnp.int32).reshape(R2, B)
    d2 = dst2.astype(jnp.int32).reshape(R2, B)

    h0p, h1p, h2p, h3p = _hists(s1a, d1p, s2, d2)
    deg_out1 = h0p.reshape(NC, H0, 1)
    deg_in1 = h1p.reshape(NC, N1P, 1)
    deg_out2 = h2p.reshape(NC, N1P, 1)
    cnt2 = h3p.reshape(NC, N2, 1)

    y0 = _prescale(x0, deg_out1)
    agg1_parts = _agg1(y0, s1a, d1p).reshape(NC, N1P, D)
    h1d, h1s = _layer1(agg1_parts, deg_in1, deg_out2, W1, b1.reshape(1, D))
    agg2_parts, nb_parts = _agg2(h1s, h1d, s2, d2)
    agg2_parts = agg2_parts.reshape(NC, N2, D)
    nb_parts = nb_parts.reshape(NC, N2, D)
    proj_h, moco_h, nbn = _feats(
        agg2_parts, nb_parts, cnt2, W2, b2.reshape(1, D),
        P1W, P1b.reshape(1, D), P2W, P2b.reshape(1, D))
    lossm = _loss(proj_h, moco_h, nbn)
    return lossm[0, 0], proj_h, moco_h


# speedup vs baseline: 1.1174x; 1.1174x over previous
import functools

import jax
import jax.numpy as jnp
from jax import lax
from jax.experimental import pallas as pl
from jax.experimental.pallas import tpu as pltpu
from jax.experimental.pallas import tpu_sc as plsc

N0, N1, N2, D = 50000, 10000, 2048, 128
E1, E2 = 160000, 32768
TEMPER, WEIGHT = 0.2, 1.0

NC, NS = 2, 16
NW = NC * NS
B = 128
R1P = 1280
R2 = E2 // B
PAD1 = R1P * B - E1

H0 = 51200
H1 = 10240
H2 = 2048
N1P = 10240

_MESH = plsc.VectorSubcoreMesh(core_axis_name="c", subcore_axis_name="s")


def _fill(ref, rows, val):
    v = jnp.full((16,), val, jnp.float32)
    if len(ref.shape) == 1:
        for i in range(rows // 16):
            ref[pl.ds(i * 16, 16)] = v
    else:
        for r in range(rows):
            for j in range(ref.shape[1] // 16):
                ref[r, pl.ds(j * 16, 16)] = v


@functools.partial(
    pl.kernel,
    out_type=(
        jax.ShapeDtypeStruct((NC * H0,), jnp.float32),
        jax.ShapeDtypeStruct((NC * H1,), jnp.float32),
        jax.ShapeDtypeStruct((NC * H1,), jnp.float32),
        jax.ShapeDtypeStruct((NC * H2,), jnp.float32),
    ),
    mesh=_MESH,
    scratch_types=[
        pltpu.VMEM((3200,), jnp.float32),
        pltpu.VMEM((B,), jnp.float32),
        pltpu.VMEM((R1P // NW, B), jnp.int32),
        pltpu.VMEM((R1P // NW, B), jnp.int32),
        pltpu.VMEM((R2 // NW, B), jnp.int32),
        pltpu.VMEM((R2 // NW, B), jnp.int32),
        pltpu.SemaphoreType.DMA,
        pltpu.VMEM_SHARED((H0,), jnp.float32),
        pltpu.VMEM_SHARED((H1,), jnp.float32),
        pltpu.VMEM_SHARED((H1,), jnp.float32),
        pltpu.VMEM_SHARED((H2,), jnp.float32),
    ],
)
def _hists(s1, d1, s2, d2, o0, o1, o2, o3, zb, ones, i1, i2, i3, i4,
           sem, h0, h1a, h1b, h3):
    c = lax.axis_index("c")
    s = lax.axis_index("s")
    wid = s * NC + c
    nj1 = R1P // NW
    nj2 = R2 // NW

    _fill(zb, 3200, 0.0)
    _fill(ones, B, 1.0)
    pltpu.sync_copy(zb.at[pl.ds(0, 3200)], h0.at[pl.ds(s * 3200, 3200)])
    pltpu.sync_copy(zb.at[pl.ds(0, 640)], h1a.at[pl.ds(s * 640, 640)])
    pltpu.sync_copy(zb.at[pl.ds(0, 640)], h1b.at[pl.ds(s * 640, 640)])
    pltpu.sync_copy(zb.at[pl.ds(0, 128)], h3.at[pl.ds(s * 128, 128)])
    pltpu.sync_copy(s1.at[pl.ds(pl.multiple_of(wid * nj1, 8), nj1)], i1)
    pltpu.sync_copy(d1.at[pl.ds(pl.multiple_of(wid * nj1, 8), nj1)], i2)
    pltpu.sync_copy(s2.at[pl.ds(pl.multiple_of(wid * nj2, 8), nj2)], i3)
    pltpu.sync_copy(d2.at[pl.ds(pl.multiple_of(wid * nj2, 8), nj2)], i4)
    plsc.subcore_barrier()

    work = ([(i1, h0, j) for j in range(nj1)]
            + [(i2, h1a, j) for j in range(nj1)]
            + [(i3, h1b, j) for j in range(nj2)]
            + [(i4, h3, j) for j in range(nj2)])
    K = 8
    for g in range(0, len(work), K):
        descs = [
            pltpu.async_copy(ones, hist.at[idx.at[j]], sem, add=True)
            for idx, hist, j in work[g:g + K]
        ]
        for dsc in descs:
            dsc.wait()
    plsc.subcore_barrier()

    pltpu.sync_copy(h0.at[pl.ds(pl.multiple_of(s * 3200, 128), 3200)],
                    o0.at[pl.ds(pl.multiple_of(c * H0 + s * 3200, 128), 3200)])
    pltpu.sync_copy(h1a.at[pl.ds(pl.multiple_of(s * 640, 128), 640)],
                    o1.at[pl.ds(pl.multiple_of(c * H1 + s * 640, 128), 640)])
    pltpu.sync_copy(h1b.at[pl.ds(pl.multiple_of(s * 640, 128), 640)],
                    o2.at[pl.ds(pl.multiple_of(c * H1 + s * 640, 128), 640)])
    pltpu.sync_copy(h3.at[pl.ds(pl.multiple_of(s * 128, 128), 128)],
                    o3.at[pl.ds(pl.multiple_of(c * H2 + s * 128, 128), 128)])


def _make_agg(n_dst, n_chunks, two_tables):
    rows_per_tile = n_dst // NS
    nj = n_chunks // NW

    n_out = 2 if two_tables else 1
    out_type = tuple(
        jax.ShapeDtypeStruct((NC * n_dst, D), jnp.float32)
        for _ in range(n_out)
    )
    if not two_tables:
        out_type = out_type[0]
    nbuf = 2
    scratch = (
        [pltpu.VMEM((nj, B), jnp.int32),
         pltpu.VMEM((nj, B), jnp.int32)]
        + [pltpu.VMEM((B, D), jnp.float32)] * (nbuf * (2 if two_tables else 1))
        + [pltpu.SemaphoreType.DMA] * (2 * nbuf)
        + [pltpu.VMEM_SHARED((n_dst, D), jnp.float32) for _ in range(n_out)]
    )

    @functools.partial(
        pl.kernel, out_type=out_type, mesh=_MESH, scratch_types=scratch
    )
    def agg(*refs):
        if two_tables:
            (ta, tb, se, de, oa, ob, idxs, idxd, ra0, ra1, rb0, rb1,
             sem0, sem1, semx0, semx1, acca, accb) = refs
            rb = (rb0, rb1)
            ra = (ra0, ra1)
            sems = (sem0, sem1)
        else:
            (ta, se, de, oa, idxs, idxd, ra0, ra1,
             sg0, sg1, ss0, ss1, acca) = refs
            ra = (ra0, ra1)
            sg = (sg0, sg1)
            ss = (ss0, ss1)
        c = lax.axis_index("c")
        s = lax.axis_index("s")
        wid = s * NC + c

        pltpu.sync_copy(se.at[pl.ds(pl.multiple_of(wid * nj, 8), nj)], idxs)
        pltpu.sync_copy(de.at[pl.ds(pl.multiple_of(wid * nj, 8), nj)], idxd)
        _fill(ra0, B, 0.0)
        for k in range(rows_per_tile // 128):
            base = pl.multiple_of(s * rows_per_tile + k * 128, 8)
            pltpu.sync_copy(ra0, acca.at[pl.ds(base, 128)])
            if two_tables:
                pltpu.sync_copy(ra0, accb.at[pl.ds(base, 128)])
        plsc.subcore_barrier()

        if two_tables:
            ga = {0: pltpu.async_copy(ta.at[idxs.at[0]], ra[0], sems[0])}
            for j in range(nj):
                ga[j].wait()
                gb = pltpu.async_copy(tb.at[idxs.at[j]], rb[j % 2], sems[j % 2])
                if j + 1 < nj:
                    ga[j + 1] = pltpu.async_copy(
                        ta.at[idxs.at[j + 1]], ra[(j + 1) % 2],
                        sems[(j + 1) % 2])
                pltpu.sync_copy(ra[j % 2], acca.at[idxd.at[j]], add=True)
                gb.wait()
                pltpu.sync_copy(rb[j % 2], accb.at[idxd.at[j]], add=True)
        else:
            ga, sc = {}, {}
            ga[0] = pltpu.async_copy(ta.at[idxs.at[0]], ra[0], sg[0])
            for j in range(nj):
                ga[j].wait()
                sc[j] = pltpu.async_copy(ra[j % 2], acca.at[idxd.at[j]],
                                         ss[j % 2], add=True)
                if j + 1 < nj:
                    if j >= 1:
                        sc[j - 1].wait()
                    ga[j + 1] = pltpu.async_copy(
                        ta.at[idxs.at[j + 1]], ra[(j + 1) % 2],
                        sg[(j + 1) % 2])
            sc[nj - 2].wait()
            sc[nj - 1].wait()
        plsc.subcore_barrier()

        src_base = pl.multiple_of(s * rows_per_tile, 8)
        dst_base = pl.multiple_of(c * n_dst + s * rows_per_tile, 8)
        pltpu.sync_copy(
            acca.at[pl.ds(src_base, rows_per_tile)],
            oa.at[pl.ds(dst_base, rows_per_tile)],
        )
        if two_tables:
            pltpu.sync_copy(
                accb.at[pl.ds(src_base, rows_per_tile)],
                ob.at[pl.ds(dst_base, rows_per_tile)],
            )

    return agg


_agg1 = _make_agg(N1P, R1P, two_tables=False)
_agg2 = _make_agg(N2, R2, two_tables=True)


def _prescale_body(x_ref, dp_ref, o_ref):
    deg = dp_ref[0] + dp_ref[1]
    rs = lax.rsqrt(jnp.maximum(deg, 1.0))
    o_ref[...] = x_ref[...] * rs


def _prescale(x0, deg_parts):
    blk = 2000
    return pl.pallas_call(
        _prescale_body,
        grid=(N0 // blk,),
        in_specs=[
            pl.BlockSpec((blk, D), lambda i: (i, 0)),
            pl.BlockSpec((NC, blk, 1), lambda i: (0, i, 0)),
        ],
        out_specs=pl.BlockSpec((blk, D), lambda i: (i, 0)),
        out_shape=jax.ShapeDtypeStruct((N0, D), jnp.float32),
    )(x0, deg_parts)


def _layer1_body(agg_ref, din_ref, dout_ref, w_ref, b_ref, h_ref, hs_ref):
    agg = agg_ref[0] + agg_ref[1]
    rs_in = lax.rsqrt(jnp.maximum(din_ref[0] + din_ref[1], 1.0))
    h = jnp.dot(agg * rs_in, w_ref[...], preferred_element_type=jnp.float32)
    h = h + b_ref[...]
    h_ref[...] = h
    rs_out = lax.rsqrt(jnp.maximum(dout_ref[0] + dout_ref[1], 1.0))
    hs_ref[...] = h * rs_out


def _layer1(agg_parts, din_parts, dout_parts, W1, b1):
    blk = 1280
    return pl.pallas_call(
        _layer1_body,
        grid=(N1P // blk,),
        in_specs=[
            pl.BlockSpec((NC, blk, D), lambda i: (0, i, 0)),
            pl.BlockSpec((NC, blk, 1), lambda i: (0, i, 0)),
            pl.BlockSpec((NC, blk, 1), lambda i: (0, i, 0)),
            pl.BlockSpec((D, D), lambda i: (0, 0)),
            pl.BlockSpec((1, D), lambda i: (0, 0)),
        ],
        out_specs=[
            pl.BlockSpec((blk, D), lambda i: (i, 0)),
            pl.BlockSpec((blk, D), lambda i: (i, 0)),
        ],
        out_shape=[
            jax.ShapeDtypeStruct((N1P, D), jnp.float32),
            jax.ShapeDtypeStruct((N1P, D), jnp.float32),
        ],
    )(agg_parts, din_parts, dout_parts, W1, b1)


def _norm_rows(x):
    n = jnp.sqrt(jnp.sum(x * x, axis=1, keepdims=True))
    return x / jnp.maximum(n, 1e-12)


def _feats_body(agg_ref, nbs_ref, cnt_ref, w2_ref, b2_ref, p1w_ref, p1b_ref,
                p2w_ref, p2b_ref, proj_ref, moco_ref, nb_ref):
    cnt = jnp.maximum(cnt_ref[0] + cnt_ref[1], 1.0)
    agg = (agg_ref[0] + agg_ref[1]) * lax.rsqrt(cnt)
    conv = jnp.dot(agg, w2_ref[...], preferred_element_type=jnp.float32)
    conv = conv + b2_ref[...]
    moco = _norm_rows(conv)
    moco_ref[...] = moco
    nb_ref[...] = _norm_rows((nbs_ref[0] + nbs_ref[1]) / cnt)
    h = jnp.maximum(
        jnp.dot(moco, p1w_ref[...], preferred_element_type=jnp.float32)
        + p1b_ref[...], 0.0)
    h = jnp.maximum(
        jnp.dot(h, p2w_ref[...], preferred_element_type=jnp.float32)
        + p2b_ref[...], 0.0)
    p = jnp.dot(h, p2w_ref[...], preferred_element_type=jnp.float32)
    p = p + p2b_ref[...]
    proj_ref[...] = _norm_rows(p)


def _feats(agg_parts, nb_parts, cnt_parts, W2, b2, P1W, P1b, P2W, P2b):
    return pl.pallas_call(
        _feats_body,
        out_shape=[
            jax.ShapeDtypeStruct((N2, D), jnp.float32),
            jax.ShapeDtypeStruct((N2, D), jnp.float32),
            jax.ShapeDtypeStruct((N2, D), jnp.float32),
        ],
    )(agg_parts, nb_parts, cnt_parts, W2, b2, P1W, P1b, P2W, P2b)


def _loss_body(p_ref, m_ref, nb_ref, o_ref):
    i = pl.program_id(0)
    blk = p_ref.shape[0]
    p = p_ref[...]

    def nce(bmat):
        sim = lax.dot_general(
            p, bmat, (((1,), (1,)), ((), ())),
            preferred_element_type=jnp.float32) / TEMPER
        e = jnp.exp(sim)
        rsum = jnp.sum(e, axis=1)
        col = lax.broadcasted_iota(jnp.int32, sim.shape, 1)
        row = lax.broadcasted_iota(jnp.int32, sim.shape, 0)
        diag = jnp.sum(jnp.where(col == row + i * blk, e, 0.0), axis=1)
        return jnp.sum(-jnp.log(diag / rsum))

    part = (nce(m_ref[...]) + WEIGHT * nce(nb_ref[...])) / N2

    @pl.when(i == 0)
    def _():
        o_ref[...] = jnp.zeros_like(o_ref)

    o_ref[...] = o_ref[...] + part


def _loss(proj, moco, nbn):
    blk = 256
    return pl.pallas_call(
        _loss_body,
        grid=(N2 // blk,),
        in_specs=[
            pl.BlockSpec((blk, D), lambda i: (i, 0)),
            pl.BlockSpec((N2, D), lambda i: (0, 0)),
            pl.BlockSpec((N2, D), lambda i: (0, 0)),
        ],
        out_specs=pl.BlockSpec((1, 1), lambda i: (0, 0)),
        out_shape=jax.ShapeDtypeStruct((1, 1), jnp.float32),
    )(proj, moco, nbn)


def kernel(x0, src1, dst1, src2, dst2, W1, b1, W2, b2, P1W, P1b, P2W, P2b):
    it = jnp.arange(PAD1, dtype=jnp.int32)
    src1i = src1.astype(jnp.int32)
    dst1i = dst1.astype(jnp.int32)
    s1h = jnp.concatenate(
        [src1i, N0 + it % (H0 - N0)]).reshape(R1P, B)
    s1a = jnp.concatenate([src1i, it % N0]).reshape(R1P, B)
    d1p = jnp.concatenate(
        [dst1i, N1 + it % (N1P - N1)]).reshape(R1P, B)
    s2 = src2.astype(jnp.int32).reshape(R2, B)
    d2 = dst2.astype(jnp.int32).reshape(R2, B)

    h0p, h1p, h2p, h3p = _hists(s1h, d1p, s2, d2)
    deg_out1 = h0p.reshape(NC, H0)[:, :N0].reshape(NC, N0, 1)
    deg_in1 = h1p.reshape(NC, N1P, 1)
    deg_out2 = h2p.reshape(NC, N1P, 1)
    cnt2 = h3p.reshape(NC, N2, 1)

    y0 = _prescale(x0, deg_out1)
    agg1_parts = _agg1(y0, s1a, d1p).reshape(NC, N1P, D)
    h1d, h1s = _layer1(agg1_parts, deg_in1, deg_out2, W1, b1.reshape(1, D))
    agg2_parts, nb_parts = _agg2(h1s, h1d, s2, d2)
    agg2_parts = agg2_parts.reshape(NC, N2, D)
    nb_parts = nb_parts.reshape(NC, N2, D)
    proj_h, moco_h, nbn = _feats(
        agg2_parts, nb_parts, cnt2, W2, b2.reshape(1, D),
        P1W, P1b.reshape(1, D), P2W, P2b.reshape(1, D))
    lossm = _loss(proj_h, moco_h, nbn)
    return lossm[0, 0], proj_h, moco_h


# speedup vs baseline: 1.1341x; 1.0150x over previous
import functools

import jax
import jax.numpy as jnp
from jax import lax
from jax.experimental import pallas as pl
from jax.experimental.pallas import tpu as pltpu
from jax.experimental.pallas import tpu_sc as plsc

N0, N1, N2, D = 50000, 10000, 2048, 128
E1, E2 = 160000, 32768
TEMPER, WEIGHT = 0.2, 1.0

NC, NS = 2, 16
NW = NC * NS
B = 128
R1P = 1280
R2 = E2 // B
PAD1 = R1P * B - E1

H0 = 51200
H1 = 10240
H2 = 2048
N1P = 10240

_MESH = plsc.VectorSubcoreMesh(core_axis_name="c", subcore_axis_name="s")


def _fill(ref, rows, val):
    v = jnp.full((16,), val, jnp.float32)
    if len(ref.shape) == 1:
        for i in range(rows // 16):
            ref[pl.ds(i * 16, 16)] = v
    else:
        for r in range(rows):
            for j in range(ref.shape[1] // 16):
                ref[r, pl.ds(j * 16, 16)] = v


@functools.partial(
    pl.kernel,
    out_type=(
        jax.ShapeDtypeStruct((NC * H0,), jnp.float32),
        jax.ShapeDtypeStruct((NC * H1,), jnp.float32),
        jax.ShapeDtypeStruct((NC * H1,), jnp.float32),
        jax.ShapeDtypeStruct((NC * H2,), jnp.float32),
    ),
    mesh=_MESH,
    scratch_types=[
        pltpu.VMEM((3200,), jnp.float32),
        pltpu.VMEM((B,), jnp.float32),
        pltpu.VMEM((R1P // NW, B), jnp.int32),
        pltpu.VMEM((R1P // NW, B), jnp.int32),
        pltpu.VMEM((R2 // NW, B), jnp.int32),
        pltpu.VMEM((R2 // NW, B), jnp.int32),
        pltpu.SemaphoreType.DMA,
        pltpu.VMEM_SHARED((H0,), jnp.float32),
        pltpu.VMEM_SHARED((H1,), jnp.float32),
        pltpu.VMEM_SHARED((H1,), jnp.float32),
        pltpu.VMEM_SHARED((H2,), jnp.float32),
    ],
)
def _hists(s1, d1, s2, d2, o0, o1, o2, o3, zb, ones, i1, i2, i3, i4,
           sem, h0, h1a, h1b, h3):
    c = lax.axis_index("c")
    s = lax.axis_index("s")
    wid = s * NC + c
    nj1 = R1P // NW
    nj2 = R2 // NW

    _fill(zb, 3200, 0.0)
    _fill(ones, B, 1.0)
    pltpu.sync_copy(zb.at[pl.ds(0, 3200)], h0.at[pl.ds(s * 3200, 3200)])
    pltpu.sync_copy(zb.at[pl.ds(0, 640)], h1a.at[pl.ds(s * 640, 640)])
    pltpu.sync_copy(zb.at[pl.ds(0, 640)], h1b.at[pl.ds(s * 640, 640)])
    pltpu.sync_copy(zb.at[pl.ds(0, 128)], h3.at[pl.ds(s * 128, 128)])
    pltpu.sync_copy(s1.at[pl.ds(pl.multiple_of(wid * nj1, 8), nj1)], i1)
    pltpu.sync_copy(d1.at[pl.ds(pl.multiple_of(wid * nj1, 8), nj1)], i2)
    pltpu.sync_copy(s2.at[pl.ds(pl.multiple_of(wid * nj2, 8), nj2)], i3)
    pltpu.sync_copy(d2.at[pl.ds(pl.multiple_of(wid * nj2, 8), nj2)], i4)
    plsc.subcore_barrier()

    work = ([(i1, h0, j) for j in range(nj1)]
            + [(i2, h1a, j) for j in range(nj1)]
            + [(i3, h1b, j) for j in range(nj2)]
            + [(i4, h3, j) for j in range(nj2)])
    K = 8
    for g in range(0, len(work), K):
        descs = [
            pltpu.async_copy(ones, hist.at[idx.at[j]], sem, add=True)
            for idx, hist, j in work[g:g + K]
        ]
        for dsc in descs:
            dsc.wait()
    plsc.subcore_barrier()

    pltpu.sync_copy(h0.at[pl.ds(pl.multiple_of(s * 3200, 128), 3200)],
                    o0.at[pl.ds(pl.multiple_of(c * H0 + s * 3200, 128), 3200)])
    pltpu.sync_copy(h1a.at[pl.ds(pl.multiple_of(s * 640, 128), 640)],
                    o1.at[pl.ds(pl.multiple_of(c * H1 + s * 640, 128), 640)])
    pltpu.sync_copy(h1b.at[pl.ds(pl.multiple_of(s * 640, 128), 640)],
                    o2.at[pl.ds(pl.multiple_of(c * H1 + s * 640, 128), 640)])
    pltpu.sync_copy(h3.at[pl.ds(pl.multiple_of(s * 128, 128), 128)],
                    o3.at[pl.ds(pl.multiple_of(c * H2 + s * 128, 128), 128)])


def _make_agg(n_dst, n_chunks, two_tables):
    rows_per_tile = n_dst // NS
    nj = n_chunks // NW

    n_out = 2 if two_tables else 1
    out_type = tuple(
        jax.ShapeDtypeStruct((NC * n_dst, D), jnp.float32)
        for _ in range(n_out)
    )
    if not two_tables:
        out_type = out_type[0]
    nbuf = 2
    scratch = (
        [pltpu.VMEM((nj, B), jnp.int32),
         pltpu.VMEM((nj, B), jnp.int32)]
        + [pltpu.VMEM((B, D), jnp.float32)] * (nbuf * (2 if two_tables else 1))
        + [pltpu.SemaphoreType.DMA] * (2 * nbuf * (2 if two_tables else 1))
        + [pltpu.VMEM_SHARED((n_dst, D), jnp.float32) for _ in range(n_out)]
    )

    @functools.partial(
        pl.kernel, out_type=out_type, mesh=_MESH, scratch_types=scratch
    )
    def agg(*refs):
        if two_tables:
            (ta, tb, se, de, oa, ob, idxs, idxd, ra0, ra1, rb0, rb1,
             sga0, sga1, sgb0, sgb1, ssa0, ssa1, ssb0, ssb1,
             acca, accb) = refs
            ra = (ra0, ra1)
            rb = (rb0, rb1)
            sga, sgb = (sga0, sga1), (sgb0, sgb1)
            ssa, ssb = (ssa0, ssa1), (ssb0, ssb1)
        else:
            (ta, se, de, oa, idxs, idxd, ra0, ra1,
             sg0, sg1, ss0, ss1, acca) = refs
            ra = (ra0, ra1)
            sg = (sg0, sg1)
            ss = (ss0, ss1)
        c = lax.axis_index("c")
        s = lax.axis_index("s")
        wid = s * NC + c

        pltpu.sync_copy(se.at[pl.ds(pl.multiple_of(wid * nj, 8), nj)], idxs)
        pltpu.sync_copy(de.at[pl.ds(pl.multiple_of(wid * nj, 8), nj)], idxd)
        _fill(ra0, B, 0.0)
        for k in range(rows_per_tile // 128):
            base = pl.multiple_of(s * rows_per_tile + k * 128, 8)
            pltpu.sync_copy(ra0, acca.at[pl.ds(base, 128)])
            if two_tables:
                pltpu.sync_copy(ra0, accb.at[pl.ds(base, 128)])
        plsc.subcore_barrier()

        if two_tables:
            ga = {0: pltpu.async_copy(ta.at[idxs.at[0]], ra[0], sga[0])}
            gb = {0: pltpu.async_copy(tb.at[idxs.at[0]], rb[0], sgb[0])}
            sca, scb = {}, {}
            for j in range(nj):
                ga[j].wait()
                sca[j] = pltpu.async_copy(ra[j % 2], acca.at[idxd.at[j]],
                                          ssa[j % 2], add=True)
                gb[j].wait()
                scb[j] = pltpu.async_copy(rb[j % 2], accb.at[idxd.at[j]],
                                          ssb[j % 2], add=True)
                if j + 1 < nj:
                    if j >= 1:
                        sca[j - 1].wait()
                        scb[j - 1].wait()
                    ga[j + 1] = pltpu.async_copy(
                        ta.at[idxs.at[j + 1]], ra[(j + 1) % 2],
                        sga[(j + 1) % 2])
                    gb[j + 1] = pltpu.async_copy(
                        tb.at[idxs.at[j + 1]], rb[(j + 1) % 2],
                        sgb[(j + 1) % 2])
            sca[nj - 2].wait()
            scb[nj - 2].wait()
            sca[nj - 1].wait()
            scb[nj - 1].wait()
        else:
            ga, sc = {}, {}
            ga[0] = pltpu.async_copy(ta.at[idxs.at[0]], ra[0], sg[0])
            for j in range(nj):
                ga[j].wait()
                sc[j] = pltpu.async_copy(ra[j % 2], acca.at[idxd.at[j]],
                                         ss[j % 2], add=True)
                if j + 1 < nj:
                    if j >= 1:
                        sc[j - 1].wait()
                    ga[j + 1] = pltpu.async_copy(
                        ta.at[idxs.at[j + 1]], ra[(j + 1) % 2],
                        sg[(j + 1) % 2])
            sc[nj - 2].wait()
            sc[nj - 1].wait()
        plsc.subcore_barrier()

        src_base = pl.multiple_of(s * rows_per_tile, 8)
        dst_base = pl.multiple_of(c * n_dst + s * rows_per_tile, 8)
        pltpu.sync_copy(
            acca.at[pl.ds(src_base, rows_per_tile)],
            oa.at[pl.ds(dst_base, rows_per_tile)],
        )
        if two_tables:
            pltpu.sync_copy(
                accb.at[pl.ds(src_base, rows_per_tile)],
                ob.at[pl.ds(dst_base, rows_per_tile)],
            )

    return agg


_agg1 = _make_agg(N1P, R1P, two_tables=False)
_agg2 = _make_agg(N2, R2, two_tables=True)


def _prescale_body(x_ref, dp_ref, o_ref):
    deg = dp_ref[0] + dp_ref[1]
    rs = lax.rsqrt(jnp.maximum(deg, 1.0))
    o_ref[...] = x_ref[...] * rs


def _prescale(x0, deg_parts):
    blk = 2000
    return pl.pallas_call(
        _prescale_body,
        grid=(N0 // blk,),
        in_specs=[
            pl.BlockSpec((blk, D), lambda i: (i, 0)),
            pl.BlockSpec((NC, blk, 1), lambda i: (0, i, 0)),
        ],
        out_specs=pl.BlockSpec((blk, D), lambda i: (i, 0)),
        out_shape=jax.ShapeDtypeStruct((N0, D), jnp.float32),
    )(x0, deg_parts)


def _layer1_body(agg_ref, din_ref, dout_ref, w_ref, b_ref, h_ref, hs_ref):
    agg = agg_ref[0] + agg_ref[1]
    rs_in = lax.rsqrt(jnp.maximum(din_ref[0] + din_ref[1], 1.0))
    h = jnp.dot(agg * rs_in, w_ref[...], preferred_element_type=jnp.float32)
    h = h + b_ref[...]
    h_ref[...] = h
    rs_out = lax.rsqrt(jnp.maximum(dout_ref[0] + dout_ref[1], 1.0))
    hs_ref[...] = h * rs_out


def _layer1(agg_parts, din_parts, dout_parts, W1, b1):
    blk = 1280
    return pl.pallas_call(
        _layer1_body,
        grid=(N1P // blk,),
        in_specs=[
            pl.BlockSpec((NC, blk, D), lambda i: (0, i, 0)),
            pl.BlockSpec((NC, blk, 1), lambda i: (0, i, 0)),
            pl.BlockSpec((NC, blk, 1), lambda i: (0, i, 0)),
            pl.BlockSpec((D, D), lambda i: (0, 0)),
            pl.BlockSpec((1, D), lambda i: (0, 0)),
        ],
        out_specs=[
            pl.BlockSpec((blk, D), lambda i: (i, 0)),
            pl.BlockSpec((blk, D), lambda i: (i, 0)),
        ],
        out_shape=[
            jax.ShapeDtypeStruct((N1P, D), jnp.float32),
            jax.ShapeDtypeStruct((N1P, D), jnp.float32),
        ],
    )(agg_parts, din_parts, dout_parts, W1, b1)


def _norm_rows(x):
    n = jnp.sqrt(jnp.sum(x * x, axis=1, keepdims=True))
    return x / jnp.maximum(n, 1e-12)


def _feats_body(agg_ref, nbs_ref, cnt_ref, w2_ref, b2_ref, p1w_ref, p1b_ref,
                p2w_ref, p2b_ref, proj_ref, moco_ref, nb_ref):
    cnt = jnp.maximum(cnt_ref[0] + cnt_ref[1], 1.0)
    agg = (agg_ref[0] + agg_ref[1]) * lax.rsqrt(cnt)
    conv = jnp.dot(agg, w2_ref[...], preferred_element_type=jnp.float32)
    conv = conv + b2_ref[...]
    moco = _norm_rows(conv)
    moco_ref[...] = moco
    nb_ref[...] = _norm_rows((nbs_ref[0] + nbs_ref[1]) / cnt)
    h = jnp.maximum(
        jnp.dot(moco, p1w_ref[...], preferred_element_type=jnp.float32)
        + p1b_ref[...], 0.0)
    h = jnp.maximum(
        jnp.dot(h, p2w_ref[...], preferred_element_type=jnp.float32)
        + p2b_ref[...], 0.0)
    p = jnp.dot(h, p2w_ref[...], preferred_element_type=jnp.float32)
    p = p + p2b_ref[...]
    proj_ref[...] = _norm_rows(p)


def _feats(agg_parts, nb_parts, cnt_parts, W2, b2, P1W, P1b, P2W, P2b):
    return pl.pallas_call(
        _feats_body,
        out_shape=[
            jax.ShapeDtypeStruct((N2, D), jnp.float32),
            jax.ShapeDtypeStruct((N2, D), jnp.float32),
            jax.ShapeDtypeStruct((N2, D), jnp.float32),
        ],
    )(agg_parts, nb_parts, cnt_parts, W2, b2, P1W, P1b, P2W, P2b)


def _loss_body(p_ref, m_ref, nb_ref, o_ref):
    i = pl.program_id(0)
    blk = p_ref.shape[0]
    p = p_ref[...]

    def nce(bmat):
        sim = lax.dot_general(
            p, bmat, (((1,), (1,)), ((), ())),
            preferred_element_type=jnp.float32) / TEMPER
        e = jnp.exp(sim)
        rsum = jnp.sum(e, axis=1)
        col = lax.broadcasted_iota(jnp.int32, sim.shape, 1)
        row = lax.broadcasted_iota(jnp.int32, sim.shape, 0)
        diag = jnp.sum(jnp.where(col == row + i * blk, e, 0.0), axis=1)
        return jnp.sum(-jnp.log(diag / rsum))

    part = (nce(m_ref[...]) + WEIGHT * nce(nb_ref[...])) / N2

    @pl.when(i == 0)
    def _():
        o_ref[...] = jnp.zeros_like(o_ref)

    o_ref[...] = o_ref[...] + part


def _loss(proj, moco, nbn):
    blk = 256
    return pl.pallas_call(
        _loss_body,
        grid=(N2 // blk,),
        in_specs=[
            pl.BlockSpec((blk, D), lambda i: (i, 0)),
            pl.BlockSpec((N2, D), lambda i: (0, 0)),
            pl.BlockSpec((N2, D), lambda i: (0, 0)),
        ],
        out_specs=pl.BlockSpec((1, 1), lambda i: (0, 0)),
        out_shape=jax.ShapeDtypeStruct((1, 1), jnp.float32),
    )(proj, moco, nbn)


def kernel(x0, src1, dst1, src2, dst2, W1, b1, W2, b2, P1W, P1b, P2W, P2b):
    it = jnp.arange(PAD1, dtype=jnp.int32)
    src1i = src1.astype(jnp.int32)
    dst1i = dst1.astype(jnp.int32)
    s1h = jnp.concatenate(
        [src1i, N0 + it % (H0 - N0)]).reshape(R1P, B)
    s1a = jnp.concatenate([src1i, it % N0]).reshape(R1P, B)
    d1p = jnp.concatenate(
        [dst1i, N1 + it % (N1P - N1)]).reshape(R1P, B)
    s2 = src2.astype(jnp.int32).reshape(R2, B)
    d2 = dst2.astype(jnp.int32).reshape(R2, B)

    h0p, h1p, h2p, h3p = _hists(s1h, d1p, s2, d2)
    deg_out1 = h0p.reshape(NC, H0)[:, :N0].reshape(NC, N0, 1)
    deg_in1 = h1p.reshape(NC, N1P, 1)
    deg_out2 = h2p.reshape(NC, N1P, 1)
    cnt2 = h3p.reshape(NC, N2, 1)

    y0 = _prescale(x0, deg_out1)
    agg1_parts = _agg1(y0, s1a, d1p).reshape(NC, N1P, D)
    h1d, h1s = _layer1(agg1_parts, deg_in1, deg_out2, W1, b1.reshape(1, D))
    agg2_parts, nb_parts = _agg2(h1s, h1d, s2, d2)
    agg2_parts = agg2_parts.reshape(NC, N2, D)
    nb_parts = nb_parts.reshape(NC, N2, D)
    proj_h, moco_h, nbn = _feats(
        agg2_parts, nb_parts, cnt2, W2, b2.reshape(1, D),
        P1W, P1b.reshape(1, D), P2W, P2b.reshape(1, D))
    lossm = _loss(proj_h, moco_h, nbn)
    return lossm[0, 0], proj_h, moco_h
